# baseline (speedup 1.0000x reference)
import functools

import jax
import jax.numpy as jnp
from jax import lax
from jax.experimental import pallas as pl
from jax.experimental.pallas import tpu as pltpu
from jax.experimental.pallas import tpu_sc as plsc

_N = 10000
_E = 320000
_F = 128
_NG = 64
_NCLS = 10
_LV = 3

_CH = 128
_NW = 32
_TPW = 80
_NBUF = 1
_HALF = 40
_EPAD = _NW * _TPW * _CH
_RPT = _N // 16

_BLK = 1000
_NBLK = _N // _BLK


def _sc_aggregate(cur, src2d, dst2d, zeros):
    mesh = plsc.VectorSubcoreMesh(core_axis_name="c", subcore_axis_name="s")

    @functools.partial(
        pl.kernel,
        out_type=jax.ShapeDtypeStruct((2, _N, _F), jnp.float32),
        mesh=mesh,
        compiler_params=pltpu.CompilerParams(use_tc_tiling_on_sc=False),
        scratch_types=[
            pltpu.VMEM((_HALF, _CH), jnp.int32),
            pltpu.VMEM((_HALF, _CH), jnp.int32),
            pltpu.VMEM((_NBUF, _CH, _F), jnp.float32),
            pltpu.SemaphoreType.DMA((_NBUF,)),
            pltpu.VMEM_SHARED((_N + 8, _F), jnp.float32),
        ],
    )
    def agg(cur_hbm, src_hbm, dst_hbm, zeros_hbm, out_hbm, sidx, didx, rowsbuf,
            sems, pool):
        rows = [rowsbuf.at[b] for b in range(_NBUF)]
        c = lax.axis_index("c")
        s = lax.axis_index("s")
        w = c * 16 + s
        pltpu.sync_copy(zeros_hbm, pool.at[pl.ds(s * _RPT, _RPT)])
        plsc.subcore_barrier()

        for h in range(_TPW // _HALF):
            base = w * _TPW + h * _HALF
            pltpu.sync_copy(src_hbm.at[pl.ds(base, _HALF)], sidx)
            pltpu.sync_copy(dst_hbm.at[pl.ds(base, _HALF)], didx)
            def body(t, carry):
                pltpu.sync_copy(cur_hbm.at[sidx.at[t]], rows[0])
                pltpu.sync_copy(rows[0], pool.at[didx.at[t]], add=True)
                return carry

            lax.fori_loop(0, _HALF, body, 0)
        plsc.subcore_barrier()
        pltpu.sync_copy(pool.at[pl.ds(s * _RPT, _RPT)],
                        out_hbm.at[c, pl.ds(s * _RPT, _RPT)])

    return agg(cur, src2d, dst2d, zeros)


def _tc_input(node_feat, W, b):
    def body(x_ref, w_ref, b_ref, o_ref):
        o_ref[...] = jnp.maximum(
            jnp.dot(x_ref[...], w_ref[...], preferred_element_type=jnp.float32)
            + b_ref[...], 0.0)

    return pl.pallas_call(
        body,
        grid=(_NBLK,),
        in_specs=[
            pl.BlockSpec((_BLK, _F), lambda i: (i, 0)),
            pl.BlockSpec((_F, _F), lambda i: (0, 0)),
            pl.BlockSpec((1, _F), lambda i: (0, 0)),
        ],
        out_specs=pl.BlockSpec((_BLK, _F), lambda i: (i, 0)),
        out_shape=jax.ShapeDtypeStruct((_N, _F), jnp.float32),
    )(node_feat, W, b.reshape(1, _F))


def _tc_layer(partials, W, b, pot):
    def body(p0_ref, p1_ref, w_ref, b_ref, pot_ref, o_ref):
        pooled = p0_ref[0] + p1_ref[0]
        o_ref[...] = jnp.maximum(
            jnp.dot(pooled, w_ref[...], preferred_element_type=jnp.float32)
            + b_ref[...] + pot_ref[...], 0.0)

    return pl.pallas_call(
        body,
        grid=(_NBLK,),
        in_specs=[
            pl.BlockSpec((1, _BLK, _F), lambda i: (0, i, 0)),
            pl.BlockSpec((1, _BLK, _F), lambda i: (1, i, 0)),
            pl.BlockSpec((_F, _F), lambda i: (0, 0)),
            pl.BlockSpec((1, _F), lambda i: (0, 0)),
            pl.BlockSpec((_BLK, _F), lambda i: (i, 0)),
        ],
        out_specs=pl.BlockSpec((_BLK, _F), lambda i: (i, 0)),
        out_shape=jax.ShapeDtypeStruct((_N, _F), jnp.float32),
    )(partials, partials, W, b.reshape(1, _F), pot)


def _tc_head(cur, gids, labels, W_out, b_out, W_h1, b_h1, W_last, b_last):
    def body(cur_ref, gid_ref, lab_ref, wo_ref, bo_ref, wh_ref, bh_ref,
             wl_ref, bl_ref, logits_ref, loss_ref, acc_ref, gp_acc):
        i = pl.program_id(0)
        oh = (gid_ref[...] ==
              lax.broadcasted_iota(jnp.int32, (_BLK, _NG), 1)).astype(jnp.float32)
        part = lax.dot_general(oh, cur_ref[...],
                               dimension_numbers=(((0,), (0,)), ((), ())),
                               preferred_element_type=jnp.float32)

        @pl.when(i == 0)
        def _():
            gp_acc[...] = part

        @pl.when(i > 0)
        def _():
            gp_acc[...] = gp_acc[...] + part

        @pl.when(i == _NBLK - 1)
        def _():
            gp = gp_acc[...]
            embed = jnp.maximum(
                jnp.dot(gp, wo_ref[...], preferred_element_type=jnp.float32)
                + bo_ref[...], 0.0)
            h = jnp.maximum(
                jnp.dot(embed, wh_ref[...], preferred_element_type=jnp.float32)
                + bh_ref[...], 0.0)
            z = (jnp.dot(h, wl_ref[...], preferred_element_type=jnp.float32)
                 + bl_ref[...])
            m = jnp.max(z, axis=1, keepdims=True)
            ls = z - (m + jnp.log(jnp.sum(jnp.exp(z - m), axis=1, keepdims=True)))
            logits_ref[...] = ls
            lab = lab_ref[...]
            cls_iota = lax.broadcasted_iota(jnp.int32, (_NG, _NCLS), 1)
            picked = jnp.sum(jnp.where(cls_iota == lab, ls, 0.0), axis=1,
                             keepdims=True)
            loss_ref[...] = -jnp.sum(picked, axis=0, keepdims=True) / _NG
            is_max = ls >= jnp.max(ls, axis=1, keepdims=True)
            pred = jnp.min(jnp.where(is_max, cls_iota, _NCLS), axis=1,
                           keepdims=True)
            acc_ref[...] = (pred == lab).astype(jnp.float32)

    return pl.pallas_call(
        body,
        grid=(_NBLK,),
        in_specs=[
            pl.BlockSpec((_BLK, _F), lambda i: (i, 0)),
            pl.BlockSpec((_BLK, 1), lambda i: (i, 0)),
            pl.BlockSpec((_NG, 1), lambda i: (0, 0)),
            pl.BlockSpec((_F, _F), lambda i: (0, 0)),
            pl.BlockSpec((1, _F), lambda i: (0, 0)),
            pl.BlockSpec((_F, _F), lambda i: (0, 0)),
            pl.BlockSpec((1, _F), lambda i: (0, 0)),
            pl.BlockSpec((_F, _NCLS), lambda i: (0, 0)),
            pl.BlockSpec((1, _NCLS), lambda i: (0, 0)),
        ],
        out_specs=[
            pl.BlockSpec((_NG, _NCLS), lambda i: (0, 0)),
            pl.BlockSpec((1, 1), lambda i: (0, 0)),
            pl.BlockSpec((_NG, 1), lambda i: (0, 0)),
        ],
        out_shape=[
            jax.ShapeDtypeStruct((_NG, _NCLS), jnp.float32),
            jax.ShapeDtypeStruct((1, 1), jnp.float32),
            jax.ShapeDtypeStruct((_NG, 1), jnp.float32),
        ],
        scratch_shapes=[pltpu.VMEM((_NG, _F), jnp.float32)],
    )(cur, gids.reshape(_N, 1), labels.reshape(_NG, 1),
      W_out, b_out.reshape(1, _F), W_h1, b_h1.reshape(1, _F),
      W_last, b_last.reshape(1, _NCLS))


def kernel(node_feat, edge_index, graph_ids, labels, W_n2l, b_n2l,
           W_conv, b_conv, W_out, b_out, W_h1, b_h1, W_last, b_last):
    pad = _EPAD - _E
    src2d = jnp.concatenate(
        [edge_index[0], jnp.zeros((pad,), jnp.int32)]).reshape(_NW * _TPW, _CH)
    dst2d = jnp.concatenate(
        [edge_index[1], jnp.full((pad,), _N, jnp.int32)]).reshape(_NW * _TPW, _CH)
    zeros = jnp.zeros((_RPT, _F), jnp.float32)

    input_pot = _tc_input(node_feat, W_n2l, b_n2l)
    cur = input_pot
    for _ in range(_LV):
        partials = _sc_aggregate(cur, src2d, dst2d, zeros)
        cur = _tc_layer(partials, W_conv, b_conv, input_pot)
    logits, loss, acc = _tc_head(cur, graph_ids, labels,
                                 W_out, b_out, W_h1, b_h1, W_last, b_last)
    return logits, loss.reshape(()), acc.reshape(_NG)


# speedup vs baseline: 2.3447x; 2.3447x over previous
import functools

import jax
import jax.numpy as jnp
from jax import lax
from jax.experimental import pallas as pl
from jax.experimental.pallas import tpu as pltpu
from jax.experimental.pallas import tpu_sc as plsc

_N = 10000
_E = 320000
_F = 128
_NG = 64
_NCLS = 10
_LV = 3

_CH = 128
_NW = 32
_TPW0 = 97
_TPW1 = 60
_NCHUNK = 16 * (_TPW0 + _TPW1)
_EPAD = _NCHUNK * _CH
_RPT = _N // 16

_BLK = 1000
_NBLK = _N // _BLK


def _sc_aggregate(cur, src2d, dst2d, zeros):
    mesh = plsc.VectorSubcoreMesh(core_axis_name="c", subcore_axis_name="s")

    @functools.partial(
        pl.kernel,
        out_type=jax.ShapeDtypeStruct((2, _N, _F), jnp.float32),
        mesh=mesh,
        compiler_params=pltpu.CompilerParams(use_tc_tiling_on_sc=False),
        scratch_types=[
            pltpu.VMEM((_TPW0, _CH), jnp.int32),
            pltpu.VMEM((_TPW0, _CH), jnp.int32),
            pltpu.VMEM((_CH, _F), jnp.float32),
            pltpu.VMEM_SHARED((_N + 8, _F), jnp.float32),
        ],
    )
    def agg(cur_hbm, src_hbm, dst_hbm, zeros_hbm, out_hbm, sidx, didx, rows,
            pool):
        c = lax.axis_index("c")
        s = lax.axis_index("s")
        pltpu.sync_copy(zeros_hbm, pool.at[pl.ds(s * _RPT, _RPT)])
        @pl.when(c == 0)
        def _():
            base = s * _TPW0
            pltpu.sync_copy(src_hbm.at[pl.ds(base, _TPW0)], sidx)
            pltpu.sync_copy(dst_hbm.at[pl.ds(base, _TPW0)], didx)

        @pl.when(c == 1)
        def _():
            base = 16 * _TPW0 + s * _TPW1
            pltpu.sync_copy(src_hbm.at[pl.ds(base, _TPW1)],
                            sidx.at[pl.ds(0, _TPW1)])
            pltpu.sync_copy(dst_hbm.at[pl.ds(base, _TPW1)],
                            didx.at[pl.ds(0, _TPW1)])
        plsc.subcore_barrier()

        def body(t, carry):
            pltpu.sync_copy(cur_hbm.at[sidx.at[t]], rows)
            pltpu.sync_copy(rows, pool.at[didx.at[t]], add=True)
            return carry

        nch = jnp.where(c == 0, _TPW0, _TPW1)
        lax.fori_loop(0, nch, body, 0)
        plsc.subcore_barrier()
        pltpu.sync_copy(pool.at[pl.ds(s * _RPT, _RPT)],
                        out_hbm.at[c, pl.ds(s * _RPT, _RPT)])

    return agg(cur, src2d, dst2d, zeros)


def _tc_input(node_feat, W, b):
    def body(x_ref, w_ref, b_ref, o_ref):
        o_ref[...] = jnp.maximum(
            jnp.dot(x_ref[...], w_ref[...], preferred_element_type=jnp.float32)
            + b_ref[...], 0.0)

    return pl.pallas_call(
        body,
        grid=(_NBLK,),
        in_specs=[
            pl.BlockSpec((_BLK, _F), lambda i: (i, 0)),
            pl.BlockSpec((_F, _F), lambda i: (0, 0)),
            pl.BlockSpec((1, _F), lambda i: (0, 0)),
        ],
        out_specs=pl.BlockSpec((_BLK, _F), lambda i: (i, 0)),
        out_shape=jax.ShapeDtypeStruct((_N, _F), jnp.float32),
    )(node_feat, W, b.reshape(1, _F))


def _tc_layer(partials, W, b, pot):
    def body(p0_ref, p1_ref, w_ref, b_ref, pot_ref, o_ref):
        pooled = p0_ref[0] + p1_ref[0]
        o_ref[...] = jnp.maximum(
            jnp.dot(pooled, w_ref[...], preferred_element_type=jnp.float32)
            + b_ref[...] + pot_ref[...], 0.0)

    return pl.pallas_call(
        body,
        grid=(_NBLK,),
        in_specs=[
            pl.BlockSpec((1, _BLK, _F), lambda i: (0, i, 0)),
            pl.BlockSpec((1, _BLK, _F), lambda i: (1, i, 0)),
            pl.BlockSpec((_F, _F), lambda i: (0, 0)),
            pl.BlockSpec((1, _F), lambda i: (0, 0)),
            pl.BlockSpec((_BLK, _F), lambda i: (i, 0)),
        ],
        out_specs=pl.BlockSpec((_BLK, _F), lambda i: (i, 0)),
        out_shape=jax.ShapeDtypeStruct((_N, _F), jnp.float32),
    )(partials, partials, W, b.reshape(1, _F), pot)


def _tc_head(cur, gids, labels, W_out, b_out, W_h1, b_h1, W_last, b_last):
    def body(cur_ref, gid_ref, lab_ref, wo_ref, bo_ref, wh_ref, bh_ref,
             wl_ref, bl_ref, logits_ref, loss_ref, acc_ref, gp_acc):
        i = pl.program_id(0)
        oh = (gid_ref[...] ==
              lax.broadcasted_iota(jnp.int32, (_BLK, _NG), 1)).astype(jnp.float32)
        part = lax.dot_general(oh, cur_ref[...],
                               dimension_numbers=(((0,), (0,)), ((), ())),
                               preferred_element_type=jnp.float32)

        @pl.when(i == 0)
        def _():
            gp_acc[...] = part

        @pl.when(i > 0)
        def _():
            gp_acc[...] = gp_acc[...] + part

        @pl.when(i == _NBLK - 1)
        def _():
            gp = gp_acc[...]
            embed = jnp.maximum(
                jnp.dot(gp, wo_ref[...], preferred_element_type=jnp.float32)
                + bo_ref[...], 0.0)
            h = jnp.maximum(
                jnp.dot(embed, wh_ref[...], preferred_element_type=jnp.float32)
                + bh_ref[...], 0.0)
            z = (jnp.dot(h, wl_ref[...], preferred_element_type=jnp.float32)
                 + bl_ref[...])
            m = jnp.max(z, axis=1, keepdims=True)
            ls = z - (m + jnp.log(jnp.sum(jnp.exp(z - m), axis=1, keepdims=True)))
            logits_ref[...] = ls
            lab = lab_ref[...]
            cls_iota = lax.broadcasted_iota(jnp.int32, (_NG, _NCLS), 1)
            picked = jnp.sum(jnp.where(cls_iota == lab, ls, 0.0), axis=1,
                             keepdims=True)
            loss_ref[...] = -jnp.sum(picked, axis=0, keepdims=True) / _NG
            is_max = ls >= jnp.max(ls, axis=1, keepdims=True)
            pred = jnp.min(jnp.where(is_max, cls_iota, _NCLS), axis=1,
                           keepdims=True)
            acc_ref[...] = (pred == lab).astype(jnp.float32)

    return pl.pallas_call(
        body,
        grid=(_NBLK,),
        in_specs=[
            pl.BlockSpec((_BLK, _F), lambda i: (i, 0)),
            pl.BlockSpec((_BLK, 1), lambda i: (i, 0)),
            pl.BlockSpec((_NG, 1), lambda i: (0, 0)),
            pl.BlockSpec((_F, _F), lambda i: (0, 0)),
            pl.BlockSpec((1, _F), lambda i: (0, 0)),
            pl.BlockSpec((_F, _F), lambda i: (0, 0)),
            pl.BlockSpec((1, _F), lambda i: (0, 0)),
            pl.BlockSpec((_F, _NCLS), lambda i: (0, 0)),
            pl.BlockSpec((1, _NCLS), lambda i: (0, 0)),
        ],
        out_specs=[
            pl.BlockSpec((_NG, _NCLS), lambda i: (0, 0)),
            pl.BlockSpec((1, 1), lambda i: (0, 0)),
            pl.BlockSpec((_NG, 1), lambda i: (0, 0)),
        ],
        out_shape=[
            jax.ShapeDtypeStruct((_NG, _NCLS), jnp.float32),
            jax.ShapeDtypeStruct((1, 1), jnp.float32),
            jax.ShapeDtypeStruct((_NG, 1), jnp.float32),
        ],
        scratch_shapes=[pltpu.VMEM((_NG, _F), jnp.float32)],
    )(cur, gids.reshape(_N, 1), labels.reshape(_NG, 1),
      W_out, b_out.reshape(1, _F), W_h1, b_h1.reshape(1, _F),
      W_last, b_last.reshape(1, _NCLS))


def kernel(node_feat, edge_index, graph_ids, labels, W_n2l, b_n2l,
           W_conv, b_conv, W_out, b_out, W_h1, b_h1, W_last, b_last):
    pad = _EPAD - _E
    src2d = jnp.concatenate(
        [edge_index[0], jnp.zeros((pad,), jnp.int32)]).reshape(_NCHUNK, _CH)
    dst2d = jnp.concatenate(
        [edge_index[1], jnp.full((pad,), _N, jnp.int32)]).reshape(_NCHUNK, _CH)
    zeros = jnp.zeros((_RPT, _F), jnp.float32)

    input_pot = _tc_input(node_feat, W_n2l, b_n2l)
    cur = input_pot
    for _ in range(_LV):
        partials = _sc_aggregate(cur, src2d, dst2d, zeros)
        cur = _tc_layer(partials, W_conv, b_conv, input_pot)
    logits, loss, acc = _tc_head(cur, graph_ids, labels,
                                 W_out, b_out, W_h1, b_h1, W_last, b_last)
    return logits, loss.reshape(()), acc.reshape(_NG)


# speedup vs baseline: 2.4280x; 1.0355x over previous
import functools

import jax
import jax.numpy as jnp
from jax import lax
from jax.experimental import pallas as pl
from jax.experimental.pallas import tpu as pltpu
from jax.experimental.pallas import tpu_sc as plsc

_N = 10000
_E = 320000
_F = 128
_NG = 64
_NCLS = 10
_LV = 3

_CH = 128
_NW = 32
_TPW0 = 93
_TPW1 = 64
_NCHUNK = 16 * (_TPW0 + _TPW1)
_EPAD = _NCHUNK * _CH
_RPT = _N // 16

_BLK = 1000
_NBLK = _N // _BLK


def _sc_aggregate(cur, src2d, dst2d, zeros):
    mesh = plsc.VectorSubcoreMesh(core_axis_name="c", subcore_axis_name="s")

    @functools.partial(
        pl.kernel,
        out_type=jax.ShapeDtypeStruct((2, _N, _F), jnp.float32),
        mesh=mesh,
        compiler_params=pltpu.CompilerParams(use_tc_tiling_on_sc=False),
        scratch_types=[
            pltpu.VMEM((_TPW0, _CH), jnp.int32),
            pltpu.VMEM((_TPW0, _CH), jnp.int32),
            pltpu.VMEM((_CH, _F), jnp.float32),
            pltpu.VMEM_SHARED((_N + 8, _F), jnp.float32),
        ],
    )
    def agg(cur_hbm, src_hbm, dst_hbm, zeros_hbm, out_hbm, sidx, didx, rows,
            pool):
        c = lax.axis_index("c")
        s = lax.axis_index("s")
        pltpu.sync_copy(zeros_hbm, pool.at[pl.ds(s * _RPT, _RPT)])
        @pl.when(c == 0)
        def _():
            base = s * _TPW0
            pltpu.sync_copy(src_hbm.at[pl.ds(base, _TPW0)], sidx)
            pltpu.sync_copy(dst_hbm.at[pl.ds(base, _TPW0)], didx)

        @pl.when(c == 1)
        def _():
            base = 16 * _TPW0 + s * _TPW1
            pltpu.sync_copy(src_hbm.at[pl.ds(base, _TPW1)],
                            sidx.at[pl.ds(0, _TPW1)])
            pltpu.sync_copy(dst_hbm.at[pl.ds(base, _TPW1)],
                            didx.at[pl.ds(0, _TPW1)])
        plsc.subcore_barrier()

        def body(t, carry):
            pltpu.sync_copy(cur_hbm.at[sidx.at[t]], rows)
            pltpu.sync_copy(rows, pool.at[didx.at[t]], add=True)
            return carry

        nch = jnp.where(c == 0, _TPW0, _TPW1)
        lax.fori_loop(0, nch, body, 0)
        plsc.subcore_barrier()
        pltpu.sync_copy(pool.at[pl.ds(s * _RPT, _RPT)],
                        out_hbm.at[c, pl.ds(s * _RPT, _RPT)])

    return agg(cur, src2d, dst2d, zeros)


def _tc_input(node_feat, W, b):
    def body(x_ref, w_ref, b_ref, o_ref):
        o_ref[...] = jnp.maximum(
            jnp.dot(x_ref[...], w_ref[...], preferred_element_type=jnp.float32)
            + b_ref[...], 0.0)

    return pl.pallas_call(
        body,
        grid=(_NBLK,),
        in_specs=[
            pl.BlockSpec((_BLK, _F), lambda i: (i, 0)),
            pl.BlockSpec((_F, _F), lambda i: (0, 0)),
            pl.BlockSpec((1, _F), lambda i: (0, 0)),
        ],
        out_specs=pl.BlockSpec((_BLK, _F), lambda i: (i, 0)),
        out_shape=jax.ShapeDtypeStruct((_N, _F), jnp.float32),
    )(node_feat, W, b.reshape(1, _F))


def _tc_layer(partials, W, b, pot):
    def body(p0_ref, p1_ref, w_ref, b_ref, pot_ref, o_ref):
        pooled = p0_ref[0] + p1_ref[0]
        o_ref[...] = jnp.maximum(
            jnp.dot(pooled, w_ref[...], preferred_element_type=jnp.float32)
            + b_ref[...] + pot_ref[...], 0.0)

    return pl.pallas_call(
        body,
        grid=(_NBLK,),
        in_specs=[
            pl.BlockSpec((1, _BLK, _F), lambda i: (0, i, 0)),
            pl.BlockSpec((1, _BLK, _F), lambda i: (1, i, 0)),
            pl.BlockSpec((_F, _F), lambda i: (0, 0)),
            pl.BlockSpec((1, _F), lambda i: (0, 0)),
            pl.BlockSpec((_BLK, _F), lambda i: (i, 0)),
        ],
        out_specs=pl.BlockSpec((_BLK, _F), lambda i: (i, 0)),
        out_shape=jax.ShapeDtypeStruct((_N, _F), jnp.float32),
    )(partials, partials, W, b.reshape(1, _F), pot)


def _tc_head(cur, gids, labels, W_out, b_out, W_h1, b_h1, W_last, b_last):
    def body(cur_ref, gid_ref, lab_ref, wo_ref, bo_ref, wh_ref, bh_ref,
             wl_ref, bl_ref, logits_ref, loss_ref, acc_ref, gp_acc):
        i = pl.program_id(0)
        oh = (gid_ref[...] ==
              lax.broadcasted_iota(jnp.int32, (_BLK, _NG), 1)).astype(jnp.float32)
        part = lax.dot_general(oh, cur_ref[...],
                               dimension_numbers=(((0,), (0,)), ((), ())),
                               preferred_element_type=jnp.float32)

        @pl.when(i == 0)
        def _():
            gp_acc[...] = part

        @pl.when(i > 0)
        def _():
            gp_acc[...] = gp_acc[...] + part

        @pl.when(i == _NBLK - 1)
        def _():
            gp = gp_acc[...]
            embed = jnp.maximum(
                jnp.dot(gp, wo_ref[...], preferred_element_type=jnp.float32)
                + bo_ref[...], 0.0)
            h = jnp.maximum(
                jnp.dot(embed, wh_ref[...], preferred_element_type=jnp.float32)
                + bh_ref[...], 0.0)
            z = (jnp.dot(h, wl_ref[...], preferred_element_type=jnp.float32)
                 + bl_ref[...])
            m = jnp.max(z, axis=1, keepdims=True)
            ls = z - (m + jnp.log(jnp.sum(jnp.exp(z - m), axis=1, keepdims=True)))
            logits_ref[...] = ls
            lab = lab_ref[...]
            cls_iota = lax.broadcasted_iota(jnp.int32, (_NG, _NCLS), 1)
            picked = jnp.sum(jnp.where(cls_iota == lab, ls, 0.0), axis=1,
                             keepdims=True)
            loss_ref[...] = -jnp.sum(picked, axis=0, keepdims=True) / _NG
            is_max = ls >= jnp.max(ls, axis=1, keepdims=True)
            pred = jnp.min(jnp.where(is_max, cls_iota, _NCLS), axis=1,
                           keepdims=True)
            acc_ref[...] = (pred == lab).astype(jnp.float32)

    return pl.pallas_call(
        body,
        grid=(_NBLK,),
        in_specs=[
            pl.BlockSpec((_BLK, _F), lambda i: (i, 0)),
            pl.BlockSpec((_BLK, 1), lambda i: (i, 0)),
            pl.BlockSpec((_NG, 1), lambda i: (0, 0)),
            pl.BlockSpec((_F, _F), lambda i: (0, 0)),
            pl.BlockSpec((1, _F), lambda i: (0, 0)),
            pl.BlockSpec((_F, _F), lambda i: (0, 0)),
            pl.BlockSpec((1, _F), lambda i: (0, 0)),
            pl.BlockSpec((_F, _NCLS), lambda i: (0, 0)),
            pl.BlockSpec((1, _NCLS), lambda i: (0, 0)),
        ],
        out_specs=[
            pl.BlockSpec((_NG, _NCLS), lambda i: (0, 0)),
            pl.BlockSpec((1, 1), lambda i: (0, 0)),
            pl.BlockSpec((_NG, 1), lambda i: (0, 0)),
        ],
        out_shape=[
            jax.ShapeDtypeStruct((_NG, _NCLS), jnp.float32),
            jax.ShapeDtypeStruct((1, 1), jnp.float32),
            jax.ShapeDtypeStruct((_NG, 1), jnp.float32),
        ],
        scratch_shapes=[pltpu.VMEM((_NG, _F), jnp.float32)],
    )(cur, gids.reshape(_N, 1), labels.reshape(_NG, 1),
      W_out, b_out.reshape(1, _F), W_h1, b_h1.reshape(1, _F),
      W_last, b_last.reshape(1, _NCLS))


def kernel(node_feat, edge_index, graph_ids, labels, W_n2l, b_n2l,
           W_conv, b_conv, W_out, b_out, W_h1, b_h1, W_last, b_last):
    pad = _EPAD - _E
    src2d = jnp.concatenate(
        [edge_index[0], jnp.zeros((pad,), jnp.int32)]).reshape(_NCHUNK, _CH)
    dst2d = jnp.concatenate(
        [edge_index[1], jnp.full((pad,), _N, jnp.int32)]).reshape(_NCHUNK, _CH)
    zeros = jnp.zeros((_RPT, _F), jnp.float32)

    input_pot = _tc_input(node_feat, W_n2l, b_n2l)
    cur = input_pot
    for _ in range(_LV):
        partials = _sc_aggregate(cur, src2d, dst2d, zeros)
        cur = _tc_layer(partials, W_conv, b_conv, input_pot)
    logits, loss, acc = _tc_head(cur, graph_ids, labels,
                                 W_out, b_out, W_h1, b_h1, W_last, b_last)
    return logits, loss.reshape(()), acc.reshape(_NG)


# speedup vs baseline: 2.5229x; 1.0391x over previous
import functools

import jax
import jax.numpy as jnp
from jax import lax
from jax.experimental import pallas as pl
from jax.experimental.pallas import tpu as pltpu
from jax.experimental.pallas import tpu_sc as plsc

_N = 10000
_E = 320000
_F = 128
_NG = 64
_NCLS = 10
_LV = 3

_CH = 96
_NW = 32
_TPW0 = 124
_TPW1 = 86
_NCHUNK = 16 * (_TPW0 + _TPW1)
_EPAD = _NCHUNK * _CH
_RPT = _N // 16

_BLK = 1000
_NBLK = _N // _BLK


def _sc_aggregate(cur, src2d, dst2d, zeros):
    mesh = plsc.VectorSubcoreMesh(core_axis_name="c", subcore_axis_name="s")

    @functools.partial(
        pl.kernel,
        out_type=jax.ShapeDtypeStruct((2, _N, _F), jnp.float32),
        mesh=mesh,
        compiler_params=pltpu.CompilerParams(use_tc_tiling_on_sc=False),
        scratch_types=[
            pltpu.VMEM((_TPW0, _CH), jnp.int32),
            pltpu.VMEM((_TPW0, _CH), jnp.int32),
            pltpu.VMEM((_CH, _F), jnp.float32),
            pltpu.VMEM((_CH, _F), jnp.float32),
            pltpu.SemaphoreType.DMA,
            pltpu.SemaphoreType.DMA,
            pltpu.VMEM_SHARED((_N + 8, _F), jnp.float32),
        ],
    )
    def agg(cur_hbm, src_hbm, dst_hbm, zeros_hbm, out_hbm, sidx, didx, rows0,
            rows1, sem0, sem1, pool):
        c = lax.axis_index("c")
        s = lax.axis_index("s")
        pltpu.sync_copy(zeros_hbm, pool.at[pl.ds(s * _RPT, _RPT)])
        @pl.when(c == 0)
        def _():
            base = s * _TPW0
            pltpu.sync_copy(src_hbm.at[pl.ds(base, _TPW0)], sidx)
            pltpu.sync_copy(dst_hbm.at[pl.ds(base, _TPW0)], didx)

        @pl.when(c == 1)
        def _():
            base = 16 * _TPW0 + s * _TPW1
            pltpu.sync_copy(src_hbm.at[pl.ds(base, _TPW1)],
                            sidx.at[pl.ds(0, _TPW1)])
            pltpu.sync_copy(dst_hbm.at[pl.ds(base, _TPW1)],
                            didx.at[pl.ds(0, _TPW1)])
        plsc.subcore_barrier()

        nch = jnp.where(c == 0, _TPW0, _TPW1)
        pltpu.async_copy(cur_hbm.at[sidx.at[0]], rows0, sem0)
        pltpu.async_copy(cur_hbm.at[sidx.at[1]], rows1, sem1)
        zdummy = zeros_hbm.at[pl.ds(0, _CH)]

        def body(i, carry):
            t0 = 2 * i
            t1 = t0 + 1
            pltpu.make_async_copy(zdummy, rows0, sem0).wait()
            pltpu.sync_copy(rows0, pool.at[didx.at[t0]], add=True)

            @pl.when(t0 + 2 < nch)
            def _():
                pltpu.async_copy(cur_hbm.at[sidx.at[t0 + 2]], rows0, sem0)

            pltpu.make_async_copy(zdummy, rows1, sem1).wait()
            pltpu.sync_copy(rows1, pool.at[didx.at[t1]], add=True)

            @pl.when(t1 + 2 < nch)
            def _():
                pltpu.async_copy(cur_hbm.at[sidx.at[t1 + 2]], rows1, sem1)
            return carry

        lax.fori_loop(0, nch // 2, body, 0)
        plsc.subcore_barrier()
        pltpu.sync_copy(pool.at[pl.ds(s * _RPT, _RPT)],
                        out_hbm.at[c, pl.ds(s * _RPT, _RPT)])

    return agg(cur, src2d, dst2d, zeros)


def _tc_input(node_feat, W, b):
    def body(x_ref, w_ref, b_ref, o_ref):
        o_ref[...] = jnp.maximum(
            jnp.dot(x_ref[...], w_ref[...], preferred_element_type=jnp.float32)
            + b_ref[...], 0.0)

    return pl.pallas_call(
        body,
        grid=(_NBLK,),
        in_specs=[
            pl.BlockSpec((_BLK, _F), lambda i: (i, 0)),
            pl.BlockSpec((_F, _F), lambda i: (0, 0)),
            pl.BlockSpec((1, _F), lambda i: (0, 0)),
        ],
        out_specs=pl.BlockSpec((_BLK, _F), lambda i: (i, 0)),
        out_shape=jax.ShapeDtypeStruct((_N, _F), jnp.float32),
    )(node_feat, W, b.reshape(1, _F))


def _tc_layer(partials, W, b, pot):
    def body(p0_ref, p1_ref, w_ref, b_ref, pot_ref, o_ref):
        pooled = p0_ref[0] + p1_ref[0]
        o_ref[...] = jnp.maximum(
            jnp.dot(pooled, w_ref[...], preferred_element_type=jnp.float32)
            + b_ref[...] + pot_ref[...], 0.0)

    return pl.pallas_call(
        body,
        grid=(_NBLK,),
        in_specs=[
            pl.BlockSpec((1, _BLK, _F), lambda i: (0, i, 0)),
            pl.BlockSpec((1, _BLK, _F), lambda i: (1, i, 0)),
            pl.BlockSpec((_F, _F), lambda i: (0, 0)),
            pl.BlockSpec((1, _F), lambda i: (0, 0)),
            pl.BlockSpec((_BLK, _F), lambda i: (i, 0)),
        ],
        out_specs=pl.BlockSpec((_BLK, _F), lambda i: (i, 0)),
        out_shape=jax.ShapeDtypeStruct((_N, _F), jnp.float32),
    )(partials, partials, W, b.reshape(1, _F), pot)


def _tc_head(cur, gids, labels, W_out, b_out, W_h1, b_h1, W_last, b_last):
    def body(cur_ref, gid_ref, lab_ref, wo_ref, bo_ref, wh_ref, bh_ref,
             wl_ref, bl_ref, logits_ref, loss_ref, acc_ref, gp_acc):
        i = pl.program_id(0)
        oh = (gid_ref[...] ==
              lax.broadcasted_iota(jnp.int32, (_BLK, _NG), 1)).astype(jnp.float32)
        part = lax.dot_general(oh, cur_ref[...],
                               dimension_numbers=(((0,), (0,)), ((), ())),
                               preferred_element_type=jnp.float32)

        @pl.when(i == 0)
        def _():
            gp_acc[...] = part

        @pl.when(i > 0)
        def _():
            gp_acc[...] = gp_acc[...] + part

        @pl.when(i == _NBLK - 1)
        def _():
            gp = gp_acc[...]
            embed = jnp.maximum(
                jnp.dot(gp, wo_ref[...], preferred_element_type=jnp.float32)
                + bo_ref[...], 0.0)
            h = jnp.maximum(
                jnp.dot(embed, wh_ref[...], preferred_element_type=jnp.float32)
                + bh_ref[...], 0.0)
            z = (jnp.dot(h, wl_ref[...], preferred_element_type=jnp.float32)
                 + bl_ref[...])
            m = jnp.max(z, axis=1, keepdims=True)
            ls = z - (m + jnp.log(jnp.sum(jnp.exp(z - m), axis=1, keepdims=True)))
            logits_ref[...] = ls
            lab = lab_ref[...]
            cls_iota = lax.broadcasted_iota(jnp.int32, (_NG, _NCLS), 1)
            picked = jnp.sum(jnp.where(cls_iota == lab, ls, 0.0), axis=1,
                             keepdims=True)
            loss_ref[...] = -jnp.sum(picked, axis=0, keepdims=True) / _NG
            is_max = ls >= jnp.max(ls, axis=1, keepdims=True)
            pred = jnp.min(jnp.where(is_max, cls_iota, _NCLS), axis=1,
                           keepdims=True)
            acc_ref[...] = (pred == lab).astype(jnp.float32)

    return pl.pallas_call(
        body,
        grid=(_NBLK,),
        in_specs=[
            pl.BlockSpec((_BLK, _F), lambda i: (i, 0)),
            pl.BlockSpec((_BLK, 1), lambda i: (i, 0)),
            pl.BlockSpec((_NG, 1), lambda i: (0, 0)),
            pl.BlockSpec((_F, _F), lambda i: (0, 0)),
            pl.BlockSpec((1, _F), lambda i: (0, 0)),
            pl.BlockSpec((_F, _F), lambda i: (0, 0)),
            pl.BlockSpec((1, _F), lambda i: (0, 0)),
            pl.BlockSpec((_F, _NCLS), lambda i: (0, 0)),
            pl.BlockSpec((1, _NCLS), lambda i: (0, 0)),
        ],
        out_specs=[
            pl.BlockSpec((_NG, _NCLS), lambda i: (0, 0)),
            pl.BlockSpec((1, 1), lambda i: (0, 0)),
            pl.BlockSpec((_NG, 1), lambda i: (0, 0)),
        ],
        out_shape=[
            jax.ShapeDtypeStruct((_NG, _NCLS), jnp.float32),
            jax.ShapeDtypeStruct((1, 1), jnp.float32),
            jax.ShapeDtypeStruct((_NG, 1), jnp.float32),
        ],
        scratch_shapes=[pltpu.VMEM((_NG, _F), jnp.float32)],
    )(cur, gids.reshape(_N, 1), labels.reshape(_NG, 1),
      W_out, b_out.reshape(1, _F), W_h1, b_h1.reshape(1, _F),
      W_last, b_last.reshape(1, _NCLS))


def kernel(node_feat, edge_index, graph_ids, labels, W_n2l, b_n2l,
           W_conv, b_conv, W_out, b_out, W_h1, b_h1, W_last, b_last):
    pad = _EPAD - _E
    src2d = jnp.concatenate(
        [edge_index[0], jnp.zeros((pad,), jnp.int32)]).reshape(_NCHUNK, _CH)
    dst2d = jnp.concatenate(
        [edge_index[1], jnp.full((pad,), _N, jnp.int32)]).reshape(_NCHUNK, _CH)
    zeros = jnp.zeros((_RPT, _F), jnp.float32)

    input_pot = _tc_input(node_feat, W_n2l, b_n2l)
    cur = input_pot
    for _ in range(_LV):
        partials = _sc_aggregate(cur, src2d, dst2d, zeros)
        cur = _tc_layer(partials, W_conv, b_conv, input_pot)
    logits, loss, acc = _tc_head(cur, graph_ids, labels,
                                 W_out, b_out, W_h1, b_h1, W_last, b_last)
    return logits, loss.reshape(()), acc.reshape(_NG)


# speedup vs baseline: 2.5889x; 1.0262x over previous
import functools

import jax
import jax.numpy as jnp
from jax import lax
from jax.experimental import pallas as pl
from jax.experimental.pallas import tpu as pltpu
from jax.experimental.pallas import tpu_sc as plsc

_N = 10000
_E = 320000
_F = 128
_NG = 64
_NCLS = 10
_LV = 3

_CH = 64
_NW = 32
_TPW0 = 189
_TPW1 = 126
_NCHUNK = 16 * (_TPW0 + _TPW1)
_EPAD = _NCHUNK * _CH
_RPT = _N // 16

_BLK = 1000
_NBLK = _N // _BLK


def _sc_aggregate(cur, src2d, dst2d, zeros):
    mesh = plsc.VectorSubcoreMesh(core_axis_name="c", subcore_axis_name="s")

    @functools.partial(
        pl.kernel,
        out_type=jax.ShapeDtypeStruct((2, _N, _F), jnp.float32),
        mesh=mesh,
        compiler_params=pltpu.CompilerParams(use_tc_tiling_on_sc=False),
        scratch_types=[
            pltpu.VMEM((_TPW0, _CH), jnp.int32),
            pltpu.VMEM((_TPW0, _CH), jnp.int32),
            pltpu.VMEM((_CH, _F), jnp.float32),
            pltpu.VMEM((_CH, _F), jnp.float32),
            pltpu.VMEM((_CH, _F), jnp.float32),
            pltpu.SemaphoreType.DMA,
            pltpu.SemaphoreType.DMA,
            pltpu.SemaphoreType.DMA,
            pltpu.VMEM_SHARED((_N + 8, _F), jnp.float32),
        ],
    )
    def agg(cur_hbm, src_hbm, dst_hbm, zeros_hbm, out_hbm, sidx, didx, rows0,
            rows1, rows2, sem0, sem1, sem2, pool):
        c = lax.axis_index("c")
        s = lax.axis_index("s")
        pltpu.sync_copy(zeros_hbm, pool.at[pl.ds(s * _RPT, _RPT)])
        @pl.when(c == 0)
        def _():
            base = s * _TPW0
            pltpu.sync_copy(src_hbm.at[pl.ds(base, _TPW0)], sidx)
            pltpu.sync_copy(dst_hbm.at[pl.ds(base, _TPW0)], didx)

        @pl.when(c == 1)
        def _():
            base = 16 * _TPW0 + s * _TPW1
            pltpu.sync_copy(src_hbm.at[pl.ds(base, _TPW1)],
                            sidx.at[pl.ds(0, _TPW1)])
            pltpu.sync_copy(dst_hbm.at[pl.ds(base, _TPW1)],
                            didx.at[pl.ds(0, _TPW1)])
        plsc.subcore_barrier()

        nch = jnp.where(c == 0, _TPW0, _TPW1)
        bufs = ((rows0, sem0), (rows1, sem1), (rows2, sem2))
        nb = len(bufs)
        for b, (rb, sb) in enumerate(bufs):
            pltpu.async_copy(cur_hbm.at[sidx.at[b]], rb, sb)
        zdummy = zeros_hbm.at[pl.ds(0, _CH)]

        def body(i, carry):
            for b, (rb, sb) in enumerate(bufs):
                t = nb * i + b
                pltpu.make_async_copy(zdummy, rb, sb).wait()
                pltpu.sync_copy(rb, pool.at[didx.at[t]], add=True)

                @pl.when(t + nb < nch)
                def _():
                    pltpu.async_copy(cur_hbm.at[sidx.at[t + nb]], rb, sb)
            return carry

        lax.fori_loop(0, nch // nb, body, 0)
        plsc.subcore_barrier()
        pltpu.sync_copy(pool.at[pl.ds(s * _RPT, _RPT)],
                        out_hbm.at[c, pl.ds(s * _RPT, _RPT)])

    return agg(cur, src2d, dst2d, zeros)


def _tc_input(node_feat, W, b):
    def body(x_ref, w_ref, b_ref, o_ref):
        o_ref[...] = jnp.maximum(
            jnp.dot(x_ref[...], w_ref[...], preferred_element_type=jnp.float32)
            + b_ref[...], 0.0)

    return pl.pallas_call(
        body,
        grid=(_NBLK,),
        in_specs=[
            pl.BlockSpec((_BLK, _F), lambda i: (i, 0)),
            pl.BlockSpec((_F, _F), lambda i: (0, 0)),
            pl.BlockSpec((1, _F), lambda i: (0, 0)),
        ],
        out_specs=pl.BlockSpec((_BLK, _F), lambda i: (i, 0)),
        out_shape=jax.ShapeDtypeStruct((_N, _F), jnp.float32),
    )(node_feat, W, b.reshape(1, _F))


def _tc_layer(partials, W, b, pot):
    def body(p0_ref, p1_ref, w_ref, b_ref, pot_ref, o_ref):
        pooled = p0_ref[0] + p1_ref[0]
        o_ref[...] = jnp.maximum(
            jnp.dot(pooled, w_ref[...], preferred_element_type=jnp.float32)
            + b_ref[...] + pot_ref[...], 0.0)

    return pl.pallas_call(
        body,
        grid=(_NBLK,),
        in_specs=[
            pl.BlockSpec((1, _BLK, _F), lambda i: (0, i, 0)),
            pl.BlockSpec((1, _BLK, _F), lambda i: (1, i, 0)),
            pl.BlockSpec((_F, _F), lambda i: (0, 0)),
            pl.BlockSpec((1, _F), lambda i: (0, 0)),
            pl.BlockSpec((_BLK, _F), lambda i: (i, 0)),
        ],
        out_specs=pl.BlockSpec((_BLK, _F), lambda i: (i, 0)),
        out_shape=jax.ShapeDtypeStruct((_N, _F), jnp.float32),
    )(partials, partials, W, b.reshape(1, _F), pot)


def _tc_head(cur, gids, labels, W_out, b_out, W_h1, b_h1, W_last, b_last):
    def body(cur_ref, gid_ref, lab_ref, wo_ref, bo_ref, wh_ref, bh_ref,
             wl_ref, bl_ref, logits_ref, loss_ref, acc_ref, gp_acc):
        i = pl.program_id(0)
        oh = (gid_ref[...] ==
              lax.broadcasted_iota(jnp.int32, (_BLK, _NG), 1)).astype(jnp.float32)
        part = lax.dot_general(oh, cur_ref[...],
                               dimension_numbers=(((0,), (0,)), ((), ())),
                               preferred_element_type=jnp.float32)

        @pl.when(i == 0)
        def _():
            gp_acc[...] = part

        @pl.when(i > 0)
        def _():
            gp_acc[...] = gp_acc[...] + part

        @pl.when(i == _NBLK - 1)
        def _():
            gp = gp_acc[...]
            embed = jnp.maximum(
                jnp.dot(gp, wo_ref[...], preferred_element_type=jnp.float32)
                + bo_ref[...], 0.0)
            h = jnp.maximum(
                jnp.dot(embed, wh_ref[...], preferred_element_type=jnp.float32)
                + bh_ref[...], 0.0)
            z = (jnp.dot(h, wl_ref[...], preferred_element_type=jnp.float32)
                 + bl_ref[...])
            m = jnp.max(z, axis=1, keepdims=True)
            ls = z - (m + jnp.log(jnp.sum(jnp.exp(z - m), axis=1, keepdims=True)))
            logits_ref[...] = ls
            lab = lab_ref[...]
            cls_iota = lax.broadcasted_iota(jnp.int32, (_NG, _NCLS), 1)
            picked = jnp.sum(jnp.where(cls_iota == lab, ls, 0.0), axis=1,
                             keepdims=True)
            loss_ref[...] = -jnp.sum(picked, axis=0, keepdims=True) / _NG
            is_max = ls >= jnp.max(ls, axis=1, keepdims=True)
            pred = jnp.min(jnp.where(is_max, cls_iota, _NCLS), axis=1,
                           keepdims=True)
            acc_ref[...] = (pred == lab).astype(jnp.float32)

    return pl.pallas_call(
        body,
        grid=(_NBLK,),
        in_specs=[
            pl.BlockSpec((_BLK, _F), lambda i: (i, 0)),
            pl.BlockSpec((_BLK, 1), lambda i: (i, 0)),
            pl.BlockSpec((_NG, 1), lambda i: (0, 0)),
            pl.BlockSpec((_F, _F), lambda i: (0, 0)),
            pl.BlockSpec((1, _F), lambda i: (0, 0)),
            pl.BlockSpec((_F, _F), lambda i: (0, 0)),
            pl.BlockSpec((1, _F), lambda i: (0, 0)),
            pl.BlockSpec((_F, _NCLS), lambda i: (0, 0)),
            pl.BlockSpec((1, _NCLS), lambda i: (0, 0)),
        ],
        out_specs=[
            pl.BlockSpec((_NG, _NCLS), lambda i: (0, 0)),
            pl.BlockSpec((1, 1), lambda i: (0, 0)),
            pl.BlockSpec((_NG, 1), lambda i: (0, 0)),
        ],
        out_shape=[
            jax.ShapeDtypeStruct((_NG, _NCLS), jnp.float32),
            jax.ShapeDtypeStruct((1, 1), jnp.float32),
            jax.ShapeDtypeStruct((_NG, 1), jnp.float32),
        ],
        scratch_shapes=[pltpu.VMEM((_NG, _F), jnp.float32)],
    )(cur, gids.reshape(_N, 1), labels.reshape(_NG, 1),
      W_out, b_out.reshape(1, _F), W_h1, b_h1.reshape(1, _F),
      W_last, b_last.reshape(1, _NCLS))


def kernel(node_feat, edge_index, graph_ids, labels, W_n2l, b_n2l,
           W_conv, b_conv, W_out, b_out, W_h1, b_h1, W_last, b_last):
    pad = _EPAD - _E
    src2d = jnp.concatenate(
        [edge_index[0], jnp.zeros((pad,), jnp.int32)]).reshape(_NCHUNK, _CH)
    dst2d = jnp.concatenate(
        [edge_index[1], jnp.full((pad,), _N, jnp.int32)]).reshape(_NCHUNK, _CH)
    zeros = jnp.zeros((_RPT, _F), jnp.float32)

    input_pot = _tc_input(node_feat, W_n2l, b_n2l)
    cur = input_pot
    for _ in range(_LV):
        partials = _sc_aggregate(cur, src2d, dst2d, zeros)
        cur = _tc_layer(partials, W_conv, b_conv, input_pot)
    logits, loss, acc = _tc_head(cur, graph_ids, labels,
                                 W_out, b_out, W_h1, b_h1, W_last, b_last)
    return logits, loss.reshape(()), acc.reshape(_NG)


# speedup vs baseline: 2.6814x; 1.0357x over previous
import functools

import jax
import jax.numpy as jnp
from jax import lax
from jax.experimental import pallas as pl
from jax.experimental.pallas import tpu as pltpu
from jax.experimental.pallas import tpu_sc as plsc

_N = 10000
_E = 320000
_F = 128
_NG = 64
_NCLS = 10
_LV = 3

_CH = 56
_NW = 32
_TPW0 = 252
_TPW1 = 108
_NCHUNK = 16 * (_TPW0 + _TPW1)
_EPAD = _NCHUNK * _CH
_RPT = _N // 16

_BLK = 1000
_NBLK = _N // _BLK


def _sc_aggregate(cur, src2d, dst2d, zeros):
    mesh = plsc.VectorSubcoreMesh(core_axis_name="c", subcore_axis_name="s")

    @functools.partial(
        pl.kernel,
        out_type=jax.ShapeDtypeStruct((2, _N, _F), jnp.float32),
        mesh=mesh,
        compiler_params=pltpu.CompilerParams(use_tc_tiling_on_sc=False),
        scratch_types=[
            pltpu.VMEM((_TPW0, _CH), jnp.int32),
            pltpu.VMEM((_TPW0, _CH), jnp.int32),
            pltpu.VMEM((_CH, _F), jnp.float32),
            pltpu.VMEM((_CH, _F), jnp.float32),
            pltpu.VMEM((_CH, _F), jnp.float32),
            pltpu.SemaphoreType.DMA,
            pltpu.SemaphoreType.DMA,
            pltpu.SemaphoreType.DMA,
            pltpu.VMEM_SHARED((_N + 8, _F), jnp.float32),
        ],
    )
    def agg(cur_hbm, src_hbm, dst_hbm, zeros_hbm, out_hbm, sidx, didx, rows0,
            rows1, rows2, sem0, sem1, sem2, pool):
        c = lax.axis_index("c")
        s = lax.axis_index("s")
        pltpu.sync_copy(zeros_hbm, pool.at[pl.ds(s * _RPT, _RPT)])
        @pl.when(c == 0)
        def _():
            base = s * _TPW0
            pltpu.sync_copy(src_hbm.at[pl.ds(base, _TPW0)], sidx)
            pltpu.sync_copy(dst_hbm.at[pl.ds(base, _TPW0)], didx)

        @pl.when(c == 1)
        def _():
            base = 16 * _TPW0 + s * _TPW1
            pltpu.sync_copy(src_hbm.at[pl.ds(base, _TPW1)],
                            sidx.at[pl.ds(0, _TPW1)])
            pltpu.sync_copy(dst_hbm.at[pl.ds(base, _TPW1)],
                            didx.at[pl.ds(0, _TPW1)])
        plsc.subcore_barrier()

        nch = jnp.where(c == 0, _TPW0, _TPW1)
        bufs = ((rows0, sem0), (rows1, sem1), (rows2, sem2))
        nb = len(bufs)
        for b, (rb, sb) in enumerate(bufs):
            pltpu.async_copy(cur_hbm.at[sidx.at[b]], rb, sb)
        zdummy = zeros_hbm.at[pl.ds(0, _CH)]

        def body(i, carry):
            for b, (rb, sb) in enumerate(bufs):
                t = nb * i + b
                pltpu.make_async_copy(zdummy, rb, sb).wait()
                pltpu.sync_copy(rb, pool.at[didx.at[t]], add=True)

                @pl.when(t + nb < nch)
                def _():
                    pltpu.async_copy(cur_hbm.at[sidx.at[t + nb]], rb, sb)
            return carry

        lax.fori_loop(0, nch // nb, body, 0)
        plsc.subcore_barrier()
        pltpu.sync_copy(pool.at[pl.ds(s * _RPT, _RPT)],
                        out_hbm.at[c, pl.ds(s * _RPT, _RPT)])

    return agg(cur, src2d, dst2d, zeros)


def _tc_input(node_feat, W, b):
    def body(x_ref, w_ref, b_ref, o_ref):
        o_ref[...] = jnp.maximum(
            jnp.dot(x_ref[...], w_ref[...], preferred_element_type=jnp.float32)
            + b_ref[...], 0.0)

    return pl.pallas_call(
        body,
        grid=(_NBLK,),
        in_specs=[
            pl.BlockSpec((_BLK, _F), lambda i: (i, 0)),
            pl.BlockSpec((_F, _F), lambda i: (0, 0)),
            pl.BlockSpec((1, _F), lambda i: (0, 0)),
        ],
        out_specs=pl.BlockSpec((_BLK, _F), lambda i: (i, 0)),
        out_shape=jax.ShapeDtypeStruct((_N, _F), jnp.float32),
    )(node_feat, W, b.reshape(1, _F))


def _tc_layer(partials, W, b, pot):
    def body(p0_ref, p1_ref, w_ref, b_ref, pot_ref, o_ref):
        pooled = p0_ref[0] + p1_ref[0]
        o_ref[...] = jnp.maximum(
            jnp.dot(pooled, w_ref[...], preferred_element_type=jnp.float32)
            + b_ref[...] + pot_ref[...], 0.0)

    return pl.pallas_call(
        body,
        grid=(_NBLK,),
        in_specs=[
            pl.BlockSpec((1, _BLK, _F), lambda i: (0, i, 0)),
            pl.BlockSpec((1, _BLK, _F), lambda i: (1, i, 0)),
            pl.BlockSpec((_F, _F), lambda i: (0, 0)),
            pl.BlockSpec((1, _F), lambda i: (0, 0)),
            pl.BlockSpec((_BLK, _F), lambda i: (i, 0)),
        ],
        out_specs=pl.BlockSpec((_BLK, _F), lambda i: (i, 0)),
        out_shape=jax.ShapeDtypeStruct((_N, _F), jnp.float32),
    )(partials, partials, W, b.reshape(1, _F), pot)


def _tc_head(cur, gids, labels, W_out, b_out, W_h1, b_h1, W_last, b_last):
    def body(cur_ref, gid_ref, lab_ref, wo_ref, bo_ref, wh_ref, bh_ref,
             wl_ref, bl_ref, logits_ref, loss_ref, acc_ref, gp_acc):
        i = pl.program_id(0)
        oh = (gid_ref[...] ==
              lax.broadcasted_iota(jnp.int32, (_BLK, _NG), 1)).astype(jnp.float32)
        part = lax.dot_general(oh, cur_ref[...],
                               dimension_numbers=(((0,), (0,)), ((), ())),
                               preferred_element_type=jnp.float32)

        @pl.when(i == 0)
        def _():
            gp_acc[...] = part

        @pl.when(i > 0)
        def _():
            gp_acc[...] = gp_acc[...] + part

        @pl.when(i == _NBLK - 1)
        def _():
            gp = gp_acc[...]
            embed = jnp.maximum(
                jnp.dot(gp, wo_ref[...], preferred_element_type=jnp.float32)
                + bo_ref[...], 0.0)
            h = jnp.maximum(
                jnp.dot(embed, wh_ref[...], preferred_element_type=jnp.float32)
                + bh_ref[...], 0.0)
            z = (jnp.dot(h, wl_ref[...], preferred_element_type=jnp.float32)
                 + bl_ref[...])
            m = jnp.max(z, axis=1, keepdims=True)
            ls = z - (m + jnp.log(jnp.sum(jnp.exp(z - m), axis=1, keepdims=True)))
            logits_ref[...] = ls
            lab = lab_ref[...]
            cls_iota = lax.broadcasted_iota(jnp.int32, (_NG, _NCLS), 1)
            picked = jnp.sum(jnp.where(cls_iota == lab, ls, 0.0), axis=1,
                             keepdims=True)
            loss_ref[...] = -jnp.sum(picked, axis=0, keepdims=True) / _NG
            is_max = ls >= jnp.max(ls, axis=1, keepdims=True)
            pred = jnp.min(jnp.where(is_max, cls_iota, _NCLS), axis=1,
                           keepdims=True)
            acc_ref[...] = (pred == lab).astype(jnp.float32)

    return pl.pallas_call(
        body,
        grid=(_NBLK,),
        in_specs=[
            pl.BlockSpec((_BLK, _F), lambda i: (i, 0)),
            pl.BlockSpec((_BLK, 1), lambda i: (i, 0)),
            pl.BlockSpec((_NG, 1), lambda i: (0, 0)),
            pl.BlockSpec((_F, _F), lambda i: (0, 0)),
            pl.BlockSpec((1, _F), lambda i: (0, 0)),
            pl.BlockSpec((_F, _F), lambda i: (0, 0)),
            pl.BlockSpec((1, _F), lambda i: (0, 0)),
            pl.BlockSpec((_F, _NCLS), lambda i: (0, 0)),
            pl.BlockSpec((1, _NCLS), lambda i: (0, 0)),
        ],
        out_specs=[
            pl.BlockSpec((_NG, _NCLS), lambda i: (0, 0)),
            pl.BlockSpec((1, 1), lambda i: (0, 0)),
            pl.BlockSpec((_NG, 1), lambda i: (0, 0)),
        ],
        out_shape=[
            jax.ShapeDtypeStruct((_NG, _NCLS), jnp.float32),
            jax.ShapeDtypeStruct((1, 1), jnp.float32),
            jax.ShapeDtypeStruct((_NG, 1), jnp.float32),
        ],
        scratch_shapes=[pltpu.VMEM((_NG, _F), jnp.float32)],
    )(cur, gids.reshape(_N, 1), labels.reshape(_NG, 1),
      W_out, b_out.reshape(1, _F), W_h1, b_h1.reshape(1, _F),
      W_last, b_last.reshape(1, _NCLS))


def kernel(node_feat, edge_index, graph_ids, labels, W_n2l, b_n2l,
           W_conv, b_conv, W_out, b_out, W_h1, b_h1, W_last, b_last):
    pad = _EPAD - _E
    src2d = jnp.concatenate(
        [edge_index[0], jnp.zeros((pad,), jnp.int32)]).reshape(_NCHUNK, _CH)
    dst2d = jnp.concatenate(
        [edge_index[1], jnp.full((pad,), _N, jnp.int32)]).reshape(_NCHUNK, _CH)
    zeros = jnp.zeros((_RPT, _F), jnp.float32)

    input_pot = _tc_input(node_feat, W_n2l, b_n2l)
    cur = input_pot
    for _ in range(_LV):
        partials = _sc_aggregate(cur, src2d, dst2d, zeros)
        cur = _tc_layer(partials, W_conv, b_conv, input_pot)
    logits, loss, acc = _tc_head(cur, graph_ids, labels,
                                 W_out, b_out, W_h1, b_h1, W_last, b_last)
    return logits, loss.reshape(()), acc.reshape(_NG)


# speedup vs baseline: 3.5074x; 1.3080x over previous
import functools

import jax
import jax.numpy as jnp
from jax import lax
from jax.experimental import pallas as pl
from jax.experimental.pallas import tpu as pltpu
from jax.experimental.pallas import tpu_sc as plsc

_N = 10000
_E = 320000
_F = 128
_NG = 64
_NCLS = 10
_LV = 3

_CH = 88
_NW = 32
_TPW0 = 162
_TPW1 = 66
_NCHUNK = 16 * (_TPW0 + _TPW1)
_EPAD = _NCHUNK * _CH
_RPT = _N // 16

_BLK = 1000
_NBLK = _N // _BLK


def _sc_aggregate(cur, src2d, dst2d, zeros):
    mesh = plsc.VectorSubcoreMesh(core_axis_name="c", subcore_axis_name="s")

    @functools.partial(
        pl.kernel,
        out_type=jax.ShapeDtypeStruct((2, _N, _F), jnp.float32),
        mesh=mesh,
        compiler_params=pltpu.CompilerParams(use_tc_tiling_on_sc=False),
        scratch_types=[
            pltpu.VMEM((_TPW0 // 2, _CH), jnp.int32),
            pltpu.VMEM((_TPW0 // 2, _CH), jnp.int32),
            pltpu.VMEM((_CH, _F), jnp.float32),
            pltpu.VMEM((_CH, _F), jnp.float32),
            pltpu.VMEM((_CH, _F), jnp.float32),
            pltpu.SemaphoreType.DMA,
            pltpu.SemaphoreType.DMA,
            pltpu.SemaphoreType.DMA,
            pltpu.VMEM_SHARED((_N + 8, _F), jnp.float32),
        ],
    )
    def agg(cur_hbm, src_hbm, dst_hbm, zeros_hbm, out_hbm, sidx, didx, rows0,
            rows1, rows2, sem0, sem1, sem2, pool):
        c = lax.axis_index("c")
        s = lax.axis_index("s")
        pltpu.sync_copy(zeros_hbm, pool.at[pl.ds(s * _RPT, _RPT)])
        plsc.subcore_barrier()

        bufs = ((rows0, sem0), (rows1, sem1), (rows2, sem2))
        nb = len(bufs)
        zdummy = zeros_hbm.at[pl.ds(0, _CH)]
        h0 = _TPW0 // 2
        h1 = _TPW1 // 2
        nch = jnp.where(c == 0, h0, h1)

        for h in range(2):
            @pl.when(c == 0)
            def _():
                base = s * _TPW0 + h * h0
                pltpu.sync_copy(src_hbm.at[pl.ds(base, h0)], sidx)
                pltpu.sync_copy(dst_hbm.at[pl.ds(base, h0)], didx)

            @pl.when(c == 1)
            def _():
                base = 16 * _TPW0 + s * _TPW1 + h * h1
                pltpu.sync_copy(src_hbm.at[pl.ds(base, h1)],
                                sidx.at[pl.ds(0, h1)])
                pltpu.sync_copy(dst_hbm.at[pl.ds(base, h1)],
                                didx.at[pl.ds(0, h1)])

            for b, (rb, sb) in enumerate(bufs):
                pltpu.async_copy(cur_hbm.at[sidx.at[b]], rb, sb)

            def body(i, carry):
                for b, (rb, sb) in enumerate(bufs):
                    t = nb * i + b
                    pltpu.make_async_copy(zdummy, rb, sb).wait()
                    pltpu.sync_copy(rb, pool.at[didx.at[t]], add=True)

                    @pl.when(t + nb < nch)
                    def _():
                        pltpu.async_copy(cur_hbm.at[sidx.at[t + nb]], rb, sb)
                return carry

            lax.fori_loop(0, nch // nb, body, 0)
        plsc.subcore_barrier()
        pltpu.sync_copy(pool.at[pl.ds(s * _RPT, _RPT)],
                        out_hbm.at[c, pl.ds(s * _RPT, _RPT)])

    return agg(cur, src2d, dst2d, zeros)


def _tc_input(node_feat, W, b):
    def body(x_ref, w_ref, b_ref, o_ref):
        o_ref[...] = jnp.maximum(
            jnp.dot(x_ref[...], w_ref[...], preferred_element_type=jnp.float32)
            + b_ref[...], 0.0)

    return pl.pallas_call(
        body,
        grid=(_NBLK,),
        in_specs=[
            pl.BlockSpec((_BLK, _F), lambda i: (i, 0)),
            pl.BlockSpec((_F, _F), lambda i: (0, 0)),
            pl.BlockSpec((1, _F), lambda i: (0, 0)),
        ],
        out_specs=pl.BlockSpec((_BLK, _F), lambda i: (i, 0)),
        out_shape=jax.ShapeDtypeStruct((_N, _F), jnp.float32),
    )(node_feat, W, b.reshape(1, _F))


def _tc_layer(partials, W, b, pot):
    def body(p0_ref, p1_ref, w_ref, b_ref, pot_ref, o_ref):
        pooled = p0_ref[0] + p1_ref[0]
        o_ref[...] = jnp.maximum(
            jnp.dot(pooled, w_ref[...], preferred_element_type=jnp.float32)
            + b_ref[...] + pot_ref[...], 0.0)

    return pl.pallas_call(
        body,
        grid=(_NBLK,),
        in_specs=[
            pl.BlockSpec((1, _BLK, _F), lambda i: (0, i, 0)),
            pl.BlockSpec((1, _BLK, _F), lambda i: (1, i, 0)),
            pl.BlockSpec((_F, _F), lambda i: (0, 0)),
            pl.BlockSpec((1, _F), lambda i: (0, 0)),
            pl.BlockSpec((_BLK, _F), lambda i: (i, 0)),
        ],
        out_specs=pl.BlockSpec((_BLK, _F), lambda i: (i, 0)),
        out_shape=jax.ShapeDtypeStruct((_N, _F), jnp.float32),
    )(partials, partials, W, b.reshape(1, _F), pot)


def _tc_head(cur, gids, labels, W_out, b_out, W_h1, b_h1, W_last, b_last):
    def body(cur_ref, gid_ref, lab_ref, wo_ref, bo_ref, wh_ref, bh_ref,
             wl_ref, bl_ref, logits_ref, loss_ref, acc_ref, gp_acc):
        i = pl.program_id(0)
        oh = (gid_ref[...] ==
              lax.broadcasted_iota(jnp.int32, (_BLK, _NG), 1)).astype(jnp.float32)
        part = lax.dot_general(oh, cur_ref[...],
                               dimension_numbers=(((0,), (0,)), ((), ())),
                               preferred_element_type=jnp.float32)

        @pl.when(i == 0)
        def _():
            gp_acc[...] = part

        @pl.when(i > 0)
        def _():
            gp_acc[...] = gp_acc[...] + part

        @pl.when(i == _NBLK - 1)
        def _():
            gp = gp_acc[...]
            embed = jnp.maximum(
                jnp.dot(gp, wo_ref[...], preferred_element_type=jnp.float32)
                + bo_ref[...], 0.0)
            h = jnp.maximum(
                jnp.dot(embed, wh_ref[...], preferred_element_type=jnp.float32)
                + bh_ref[...], 0.0)
            z = (jnp.dot(h, wl_ref[...], preferred_element_type=jnp.float32)
                 + bl_ref[...])
            m = jnp.max(z, axis=1, keepdims=True)
            ls = z - (m + jnp.log(jnp.sum(jnp.exp(z - m), axis=1, keepdims=True)))
            logits_ref[...] = ls
            lab = lab_ref[...]
            cls_iota = lax.broadcasted_iota(jnp.int32, (_NG, _NCLS), 1)
            picked = jnp.sum(jnp.where(cls_iota == lab, ls, 0.0), axis=1,
                             keepdims=True)
            loss_ref[...] = -jnp.sum(picked, axis=0, keepdims=True) / _NG
            is_max = ls >= jnp.max(ls, axis=1, keepdims=True)
            pred = jnp.min(jnp.where(is_max, cls_iota, _NCLS), axis=1,
                           keepdims=True)
            acc_ref[...] = (pred == lab).astype(jnp.float32)

    return pl.pallas_call(
        body,
        grid=(_NBLK,),
        in_specs=[
            pl.BlockSpec((_BLK, _F), lambda i: (i, 0)),
            pl.BlockSpec((_BLK, 1), lambda i: (i, 0)),
            pl.BlockSpec((_NG, 1), lambda i: (0, 0)),
            pl.BlockSpec((_F, _F), lambda i: (0, 0)),
            pl.BlockSpec((1, _F), lambda i: (0, 0)),
            pl.BlockSpec((_F, _F), lambda i: (0, 0)),
            pl.BlockSpec((1, _F), lambda i: (0, 0)),
            pl.BlockSpec((_F, _NCLS), lambda i: (0, 0)),
            pl.BlockSpec((1, _NCLS), lambda i: (0, 0)),
        ],
        out_specs=[
            pl.BlockSpec((_NG, _NCLS), lambda i: (0, 0)),
            pl.BlockSpec((1, 1), lambda i: (0, 0)),
            pl.BlockSpec((_NG, 1), lambda i: (0, 0)),
        ],
        out_shape=[
            jax.ShapeDtypeStruct((_NG, _NCLS), jnp.float32),
            jax.ShapeDtypeStruct((1, 1), jnp.float32),
            jax.ShapeDtypeStruct((_NG, 1), jnp.float32),
        ],
        scratch_shapes=[pltpu.VMEM((_NG, _F), jnp.float32)],
    )(cur, gids.reshape(_N, 1), labels.reshape(_NG, 1),
      W_out, b_out.reshape(1, _F), W_h1, b_h1.reshape(1, _F),
      W_last, b_last.reshape(1, _NCLS))


def kernel(node_feat, edge_index, graph_ids, labels, W_n2l, b_n2l,
           W_conv, b_conv, W_out, b_out, W_h1, b_h1, W_last, b_last):
    pad = _EPAD - _E
    src2d = jnp.concatenate(
        [edge_index[0], jnp.zeros((pad,), jnp.int32)]).reshape(_NCHUNK, _CH)
    dst2d = jnp.concatenate(
        [edge_index[1], jnp.full((pad,), _N, jnp.int32)]).reshape(_NCHUNK, _CH)
    zeros = jnp.zeros((_RPT, _F), jnp.float32)

    input_pot = _tc_input(node_feat, W_n2l, b_n2l)
    cur = input_pot
    for _ in range(_LV):
        partials = _sc_aggregate(cur, src2d, dst2d, zeros)
        cur = _tc_layer(partials, W_conv, b_conv, input_pot)
    logits, loss, acc = _tc_head(cur, graph_ids, labels,
                                 W_out, b_out, W_h1, b_h1, W_last, b_last)
    return logits, loss.reshape(()), acc.reshape(_NG)


# speedup vs baseline: 3.5953x; 1.0251x over previous
import functools

import jax
import jax.numpy as jnp
from jax import lax
from jax.experimental import pallas as pl
from jax.experimental.pallas import tpu as pltpu
from jax.experimental.pallas import tpu_sc as plsc

_N = 10000
_E = 320000
_F = 128
_NG = 64
_NCLS = 10
_LV = 3

_CH = 88
_NW = 32
_TPW0 = 156
_TPW1 = 72
_NCHUNK = 16 * (_TPW0 + _TPW1)
_EPAD = _NCHUNK * _CH
_RPT = _N // 16

_BLK = 1000
_NBLK = _N // _BLK


def _sc_aggregate(cur, src2d, dst2d, zeros):
    mesh = plsc.VectorSubcoreMesh(core_axis_name="c", subcore_axis_name="s")

    @functools.partial(
        pl.kernel,
        out_type=jax.ShapeDtypeStruct((2, _N, _F), jnp.float32),
        mesh=mesh,
        compiler_params=pltpu.CompilerParams(use_tc_tiling_on_sc=False),
        scratch_types=[
            pltpu.VMEM((_TPW0 // 2, _CH), jnp.int32),
            pltpu.VMEM((_TPW0 // 2, _CH), jnp.int32),
            pltpu.VMEM((_CH, _F), jnp.float32),
            pltpu.VMEM((_CH, _F), jnp.float32),
            pltpu.VMEM((_CH, _F), jnp.float32),
            pltpu.SemaphoreType.DMA,
            pltpu.SemaphoreType.DMA,
            pltpu.SemaphoreType.DMA,
            pltpu.VMEM_SHARED((_N + 8, _F), jnp.float32),
        ],
    )
    def agg(cur_hbm, src_hbm, dst_hbm, zeros_hbm, out_hbm, sidx, didx, rows0,
            rows1, rows2, sem0, sem1, sem2, pool):
        c = lax.axis_index("c")
        s = lax.axis_index("s")
        pltpu.sync_copy(zeros_hbm, pool.at[pl.ds(s * _RPT, _RPT)])
        plsc.subcore_barrier()

        bufs = ((rows0, sem0), (rows1, sem1), (rows2, sem2))
        nb = len(bufs)
        zdummy = zeros_hbm.at[pl.ds(0, _CH)]
        h0 = _TPW0 // 2
        h1 = _TPW1 // 2
        nch = jnp.where(c == 0, h0, h1)

        for h in range(2):
            @pl.when(c == 0)
            def _():
                base = s * _TPW0 + h * h0
                pltpu.sync_copy(src_hbm.at[pl.ds(base, h0)], sidx)
                pltpu.sync_copy(dst_hbm.at[pl.ds(base, h0)], didx)

            @pl.when(c == 1)
            def _():
                base = 16 * _TPW0 + s * _TPW1 + h * h1
                pltpu.sync_copy(src_hbm.at[pl.ds(base, h1)],
                                sidx.at[pl.ds(0, h1)])
                pltpu.sync_copy(dst_hbm.at[pl.ds(base, h1)],
                                didx.at[pl.ds(0, h1)])

            for b, (rb, sb) in enumerate(bufs):
                pltpu.async_copy(cur_hbm.at[sidx.at[b]], rb, sb)

            def body(i, carry):
                for b, (rb, sb) in enumerate(bufs):
                    t = nb * i + b
                    pltpu.make_async_copy(zdummy, rb, sb).wait()
                    pltpu.sync_copy(rb, pool.at[didx.at[t]], add=True)

                    @pl.when(t + nb < nch)
                    def _():
                        pltpu.async_copy(cur_hbm.at[sidx.at[t + nb]], rb, sb)
                return carry

            lax.fori_loop(0, nch // nb, body, 0)
        plsc.subcore_barrier()
        pltpu.sync_copy(pool.at[pl.ds(s * _RPT, _RPT)],
                        out_hbm.at[c, pl.ds(s * _RPT, _RPT)])

    return agg(cur, src2d, dst2d, zeros)


def _tc_input(node_feat, W, b):
    def body(x_ref, w_ref, b_ref, o_ref):
        o_ref[...] = jnp.maximum(
            jnp.dot(x_ref[...], w_ref[...], preferred_element_type=jnp.float32)
            + b_ref[...], 0.0)

    return pl.pallas_call(
        body,
        grid=(_NBLK,),
        in_specs=[
            pl.BlockSpec((_BLK, _F), lambda i: (i, 0)),
            pl.BlockSpec((_F, _F), lambda i: (0, 0)),
            pl.BlockSpec((1, _F), lambda i: (0, 0)),
        ],
        out_specs=pl.BlockSpec((_BLK, _F), lambda i: (i, 0)),
        out_shape=jax.ShapeDtypeStruct((_N, _F), jnp.float32),
    )(node_feat, W, b.reshape(1, _F))


def _tc_layer(partials, W, b, pot):
    def body(p0_ref, p1_ref, w_ref, b_ref, pot_ref, o_ref):
        pooled = p0_ref[0] + p1_ref[0]
        o_ref[...] = jnp.maximum(
            jnp.dot(pooled, w_ref[...], preferred_element_type=jnp.float32)
            + b_ref[...] + pot_ref[...], 0.0)

    return pl.pallas_call(
        body,
        grid=(_NBLK,),
        in_specs=[
            pl.BlockSpec((1, _BLK, _F), lambda i: (0, i, 0)),
            pl.BlockSpec((1, _BLK, _F), lambda i: (1, i, 0)),
            pl.BlockSpec((_F, _F), lambda i: (0, 0)),
            pl.BlockSpec((1, _F), lambda i: (0, 0)),
            pl.BlockSpec((_BLK, _F), lambda i: (i, 0)),
        ],
        out_specs=pl.BlockSpec((_BLK, _F), lambda i: (i, 0)),
        out_shape=jax.ShapeDtypeStruct((_N, _F), jnp.float32),
    )(partials, partials, W, b.reshape(1, _F), pot)


def _tc_head(cur, gids, labels, W_out, b_out, W_h1, b_h1, W_last, b_last):
    def body(cur_ref, gid_ref, lab_ref, wo_ref, bo_ref, wh_ref, bh_ref,
             wl_ref, bl_ref, logits_ref, loss_ref, acc_ref, gp_acc):
        i = pl.program_id(0)
        oh = (gid_ref[...] ==
              lax.broadcasted_iota(jnp.int32, (_BLK, _NG), 1)).astype(jnp.float32)
        part = lax.dot_general(oh, cur_ref[...],
                               dimension_numbers=(((0,), (0,)), ((), ())),
                               preferred_element_type=jnp.float32)

        @pl.when(i == 0)
        def _():
            gp_acc[...] = part

        @pl.when(i > 0)
        def _():
            gp_acc[...] = gp_acc[...] + part

        @pl.when(i == _NBLK - 1)
        def _():
            gp = gp_acc[...]
            embed = jnp.maximum(
                jnp.dot(gp, wo_ref[...], preferred_element_type=jnp.float32)
                + bo_ref[...], 0.0)
            h = jnp.maximum(
                jnp.dot(embed, wh_ref[...], preferred_element_type=jnp.float32)
                + bh_ref[...], 0.0)
            z = (jnp.dot(h, wl_ref[...], preferred_element_type=jnp.float32)
                 + bl_ref[...])
            m = jnp.max(z, axis=1, keepdims=True)
            ls = z - (m + jnp.log(jnp.sum(jnp.exp(z - m), axis=1, keepdims=True)))
            logits_ref[...] = ls
            lab = lab_ref[...]
            cls_iota = lax.broadcasted_iota(jnp.int32, (_NG, _NCLS), 1)
            picked = jnp.sum(jnp.where(cls_iota == lab, ls, 0.0), axis=1,
                             keepdims=True)
            loss_ref[...] = -jnp.sum(picked, axis=0, keepdims=True) / _NG
            is_max = ls >= jnp.max(ls, axis=1, keepdims=True)
            pred = jnp.min(jnp.where(is_max, cls_iota, _NCLS), axis=1,
                           keepdims=True)
            acc_ref[...] = (pred == lab).astype(jnp.float32)

    return pl.pallas_call(
        body,
        grid=(_NBLK,),
        in_specs=[
            pl.BlockSpec((_BLK, _F), lambda i: (i, 0)),
            pl.BlockSpec((_BLK, 1), lambda i: (i, 0)),
            pl.BlockSpec((_NG, 1), lambda i: (0, 0)),
            pl.BlockSpec((_F, _F), lambda i: (0, 0)),
            pl.BlockSpec((1, _F), lambda i: (0, 0)),
            pl.BlockSpec((_F, _F), lambda i: (0, 0)),
            pl.BlockSpec((1, _F), lambda i: (0, 0)),
            pl.BlockSpec((_F, _NCLS), lambda i: (0, 0)),
            pl.BlockSpec((1, _NCLS), lambda i: (0, 0)),
        ],
        out_specs=[
            pl.BlockSpec((_NG, _NCLS), lambda i: (0, 0)),
            pl.BlockSpec((1, 1), lambda i: (0, 0)),
            pl.BlockSpec((_NG, 1), lambda i: (0, 0)),
        ],
        out_shape=[
            jax.ShapeDtypeStruct((_NG, _NCLS), jnp.float32),
            jax.ShapeDtypeStruct((1, 1), jnp.float32),
            jax.ShapeDtypeStruct((_NG, 1), jnp.float32),
        ],
        scratch_shapes=[pltpu.VMEM((_NG, _F), jnp.float32)],
    )(cur, gids.reshape(_N, 1), labels.reshape(_NG, 1),
      W_out, b_out.reshape(1, _F), W_h1, b_h1.reshape(1, _F),
      W_last, b_last.reshape(1, _NCLS))


def kernel(node_feat, edge_index, graph_ids, labels, W_n2l, b_n2l,
           W_conv, b_conv, W_out, b_out, W_h1, b_h1, W_last, b_last):
    pad = _EPAD - _E
    src2d = jnp.concatenate(
        [edge_index[0], jnp.zeros((pad,), jnp.int32)]).reshape(_NCHUNK, _CH)
    dst2d = jnp.concatenate(
        [edge_index[1], jnp.full((pad,), _N, jnp.int32)]).reshape(_NCHUNK, _CH)
    zeros = jnp.zeros((_RPT, _F), jnp.float32)

    input_pot = _tc_input(node_feat, W_n2l, b_n2l)
    cur = input_pot
    for _ in range(_LV):
        partials = _sc_aggregate(cur, src2d, dst2d, zeros)
        cur = _tc_layer(partials, W_conv, b_conv, input_pot)
    logits, loss, acc = _tc_head(cur, graph_ids, labels,
                                 W_out, b_out, W_h1, b_h1, W_last, b_last)
    return logits, loss.reshape(()), acc.reshape(_NG)
